# baseline (speedup 1.0000x reference)
import functools

import jax
import jax.numpy as jnp
from jax import lax
from jax.experimental import pallas as pl
from jax.experimental.pallas import tpu as pltpu
from jax.experimental.pallas import tpu_sc as plsc

_NUM_FIELDS = 26
_VOCAB = 100000
_EMB_DIM = 32
_LANES = 16
_HB = 8192
_H0 = 50048
_H1 = _VOCAB - _H0

_info = plsc.get_sparse_core_info()
_NC, _NS = _info.num_cores, _info.num_subcores
_NW = _NC * _NS


def _make_sc_gather(batch: int):
  n_hb = batch // _HB
  assert batch % _HB == 0 and _NW == _EMB_DIM

  mesh = plsc.VectorSubcoreMesh(core_axis_name="c", subcore_axis_name="s")

  @functools.partial(
      pl.kernel,
      mesh=mesh,
      out_type=jax.ShapeDtypeStruct((_NUM_FIELDS * _EMB_DIM, batch), jnp.float32),
      compiler_params=pltpu.CompilerParams(
          use_tc_tiling_on_sc=True, needs_layout_passes=False),
      scratch_types=[
          pltpu.VMEM((_H0,), jnp.float32),
          pltpu.VMEM((_H1,), jnp.float32),
          pltpu.VMEM((_HB,), jnp.int32),
          pltpu.VMEM((_HB,), jnp.int32),
          pltpu.VMEM((_HB,), jnp.float32),
          pltpu.SemaphoreType.DMA,
          pltpu.SemaphoreType.DMA,
          pltpu.SemaphoreType.DMA,
          pltpu.SemaphoreType.DMA,
          pltpu.SemaphoreType.DMA,
      ],
  )
  def gather_kernel(xt_hbm, tt_hbm, out_hbm,
                    rh0, rh1, iv0, iv1, out_v, sr0, sr1, si0, si1, so):
    w = lax.axis_index("s") * _NC + lax.axis_index("c")
    idx_v = (iv0, iv1)
    si = (si0, si1)
    ji = jax.lax.broadcasted_iota(jnp.int32, (_LANES,), 0)
    n_tasks = _NUM_FIELDS * n_hb

    def idx_slice(u):
      f, hb = divmod(u, n_hb)
      return xt_hbm.at[f, pl.ds(hb * _HB, _HB)]

    def row_copy(f, h):
      if h == 0:
        return pltpu.async_copy(tt_hbm.at[f, w, pl.ds(0, _H0)], rh0, sr0)
      return pltpu.async_copy(tt_hbm.at[f, w, pl.ds(_H0, _H1)], rh1, sr1)

    def do_pass(h, ib):
      @plsc.parallel_loop(0, _HB, step=_LANES, unroll=4)
      def _g(j):
        iv = ib[pl.ds(j, _LANES)]
        if h == 0:
          m = iv < _H0
          g = plsc.load_gather(rh0, [iv], mask=m)
        else:
          m = iv >= _H0
          g = plsc.load_gather(rh1, [iv - _H0], mask=m)
        plsc.store_scatter(out_v, [ji + j], g, mask=m)

    row_d = [row_copy(0, 0), row_copy(0, 1)]
    row_waited = [False, False]
    idx_d = [None, None]
    idx_d[0] = pltpu.async_copy(idx_slice(0), idx_v[0], si[0])
    out_d = None

    for u in range(n_tasks):
      b, nb = u % 2, (u + 1) % 2
      f, hb = divmod(u, n_hb)
      if u + 1 < n_tasks:
        idx_d[nb] = pltpu.async_copy(idx_slice(u + 1), idx_v[nb], si[nb])
      idx_d[b].wait()
      if out_d is not None:
        out_d.wait()
      for h in (0, 1):
        if not row_waited[h]:
          row_d[h].wait()
          row_waited[h] = True
        do_pass(h, idx_v[b])
        if hb == n_hb - 1:
          if f + 1 < _NUM_FIELDS:
            row_d[h] = row_copy(f + 1, h)
          row_waited[h] = False
      out_d = pltpu.async_copy(
          out_v, out_hbm.at[f * _EMB_DIM + w, pl.ds(hb * _HB, _HB)], so)
    out_d.wait()

  return gather_kernel


def kernel(x, tables):
  batch = x.shape[0]
  x_t = x.T.astype(jnp.int32)
  tab_t = tables.transpose(0, 2, 1)
  out_t = _make_sc_gather(batch)(x_t, tab_t)
  return out_t.T


# speedup vs baseline: 1.1597x; 1.1597x over previous
import functools

import jax
import jax.numpy as jnp
from jax import lax
from jax.experimental import pallas as pl
from jax.experimental.pallas import tpu as pltpu
from jax.experimental.pallas import tpu_sc as plsc

_NUM_FIELDS = 26
_VOCAB = 100000
_EMB_DIM = 32
_LANES = 16
_BCHUNK = 4096

_info = plsc.get_sparse_core_info()
_NC, _NS = _info.num_cores, _info.num_subcores
_NW = _NC * _NS


def _make_sc_gather(batch: int):
  n_bchunks = batch // _BCHUNK
  assert batch % _BCHUNK == 0 and _NW == _EMB_DIM

  mesh = plsc.VectorSubcoreMesh(core_axis_name="c", subcore_axis_name="s")

  @functools.partial(
      pl.kernel,
      mesh=mesh,
      out_type=jax.ShapeDtypeStruct((_NUM_FIELDS * _EMB_DIM, batch), jnp.float32),
      compiler_params=pltpu.CompilerParams(
          use_tc_tiling_on_sc=True, needs_layout_passes=False),
      scratch_types=[
          pltpu.VMEM((_VOCAB,), jnp.float32),
          pltpu.VMEM((_BCHUNK,), jnp.int32),
          pltpu.VMEM((_BCHUNK,), jnp.int32),
          pltpu.VMEM((_BCHUNK,), jnp.float32),
          pltpu.VMEM((_BCHUNK,), jnp.float32),
          pltpu.SemaphoreType.DMA,
          pltpu.SemaphoreType.DMA,
          pltpu.SemaphoreType.DMA,
          pltpu.SemaphoreType.DMA,
      ],
  )
  def gather_kernel(xt_hbm, tt_hbm, out_hbm,
                    row_v, iv0, iv1, ov0, ov1, si0, si1, sw0, sw1):
    w = lax.axis_index("s") * _NC + lax.axis_index("c")
    idx_v = (iv0, iv1)
    out_v = (ov0, ov1)
    si = (si0, si1)
    sw = (sw0, sw1)
    n_tasks = _NUM_FIELDS * n_bchunks

    def fld(f):
      return lax.rem(f + w, _NUM_FIELDS)

    def idx_slice(t):
      f, bc = divmod(t, n_bchunks)
      return xt_hbm.at[fld(f), pl.ds(bc * _BCHUNK, _BCHUNK)]

    write_d = [None, None]
    idx_d = [None, None]
    idx_d[0] = pltpu.async_copy(idx_slice(0), idx_v[0], si[0])
    for t in range(n_tasks):
      b, nb = t % 2, (t + 1) % 2
      f, bc = divmod(t, n_bchunks)
      if t + 1 < n_tasks:
        idx_d[nb] = pltpu.async_copy(idx_slice(t + 1), idx_v[nb], si[nb])
      if bc == 0:
        pltpu.sync_copy(tt_hbm.at[fld(f), w], row_v)
      idx_d[b].wait()
      if write_d[b] is not None:
        write_d[b].wait()

      @plsc.parallel_loop(0, _BCHUNK, step=_LANES, unroll=4)
      def _g(j):
        s = pl.ds(j, _LANES)
        out_v[b][s] = plsc.load_gather(row_v, [idx_v[b][s]])

      write_d[b] = pltpu.async_copy(
          out_v[b], out_hbm.at[fld(f) * _EMB_DIM + w, pl.ds(bc * _BCHUNK, _BCHUNK)],
          sw[b])
    for d in write_d:
      if d is not None:
        d.wait()

  return gather_kernel


def kernel(x, tables):
  batch = x.shape[0]
  x_t = x.T.astype(jnp.int32)
  tab_t = tables.transpose(0, 2, 1)
  out_t = _make_sc_gather(batch)(x_t, tab_t)
  return out_t.T


# speedup vs baseline: 1.1601x; 1.0003x over previous
import functools

import jax
import jax.numpy as jnp
from jax import lax
from jax.experimental import pallas as pl
from jax.experimental.pallas import tpu as pltpu
from jax.experimental.pallas import tpu_sc as plsc

_NUM_FIELDS = 26
_VOCAB = 100000
_EMB_DIM = 32
_LANES = 16
_BCHUNK = 4096

_info = plsc.get_sparse_core_info()
_NC, _NS = _info.num_cores, _info.num_subcores
_NW = _NC * _NS


def _make_sc_gather(batch: int):
  n_bchunks = batch // _BCHUNK
  assert batch % _BCHUNK == 0 and _NW == _EMB_DIM

  mesh = plsc.VectorSubcoreMesh(core_axis_name="c", subcore_axis_name="s")

  @functools.partial(
      pl.kernel,
      mesh=mesh,
      out_type=jax.ShapeDtypeStruct((_NUM_FIELDS * _EMB_DIM, batch), jnp.float32),
      compiler_params=pltpu.CompilerParams(
          use_tc_tiling_on_sc=True, needs_layout_passes=False),
      scratch_types=[
          pltpu.VMEM((_VOCAB,), jnp.float32),
          pltpu.VMEM((_BCHUNK,), jnp.int32),
          pltpu.VMEM((_BCHUNK,), jnp.int32),
          pltpu.VMEM((_BCHUNK,), jnp.float32),
          pltpu.VMEM((_BCHUNK,), jnp.float32),
          pltpu.SemaphoreType.DMA,
          pltpu.SemaphoreType.DMA,
          pltpu.SemaphoreType.DMA,
          pltpu.SemaphoreType.DMA,
      ],
  )
  def gather_kernel(xt_hbm, tt_hbm, out_hbm,
                    row_v, iv0, iv1, ov0, ov1, si0, si1, sw0, sw1):
    w = lax.axis_index("s") * _NC + lax.axis_index("c")
    idx_v = (iv0, iv1)
    out_v = (ov0, ov1)
    si = (si0, si1)
    sw = (sw0, sw1)
    n_tasks = _NUM_FIELDS * n_bchunks

    def fld(f):
      return lax.rem(f + w, _NUM_FIELDS)

    def idx_slice(t):
      f, bc = divmod(t, n_bchunks)
      return xt_hbm.at[fld(f), pl.ds(bc * _BCHUNK, _BCHUNK)]

    write_d = [None, None]
    idx_d = [None, None]
    idx_d[0] = pltpu.async_copy(idx_slice(0), idx_v[0], si[0])
    for t in range(n_tasks):
      b, nb = t % 2, (t + 1) % 2
      f, bc = divmod(t, n_bchunks)
      if t + 1 < n_tasks:
        idx_d[nb] = pltpu.async_copy(idx_slice(t + 1), idx_v[nb], si[nb])
      if bc == 0:
        pltpu.sync_copy(tt_hbm.at[fld(f), w], row_v)
      idx_d[b].wait()
      if write_d[b] is not None:
        write_d[b].wait()

      @plsc.parallel_loop(0, _BCHUNK, step=_LANES, unroll=8)
      def _g(j):
        s = pl.ds(j, _LANES)
        out_v[b][s] = plsc.load_gather(row_v, [idx_v[b][s]])

      write_d[b] = pltpu.async_copy(
          out_v[b], out_hbm.at[fld(f) * _EMB_DIM + w, pl.ds(bc * _BCHUNK, _BCHUNK)],
          sw[b])
    for d in write_d:
      if d is not None:
        d.wait()

  return gather_kernel


def kernel(x, tables):
  batch = x.shape[0]
  x_t = x.T.astype(jnp.int32)
  tab_t = tables.transpose(0, 2, 1)
  out_t = _make_sc_gather(batch)(x_t, tab_t)
  return out_t.T
